# baseline (speedup 1.0000x reference)
import functools

import numpy as np
import jax
import jax.numpy as jnp
from jax import lax
from jax.experimental import pallas as pl
from jax.experimental.pallas import tpu as pltpu
from jax.experimental.pallas import tpu_sc as plsc

_PATCH = 7
_DILATION = 2
_NC, _NS = 2, 16
_NW = _NC * _NS

_half = (_PATCH - 1) // 2


def _rows_tc_kernel(w_ref, idx_ref, out_ref, *, h_feat, w_feat):
    w = w_ref[0, 0]
    idx_f = idx_ref[...].astype(jnp.float32)
    y_pix = jnp.floor(idx_f / w)
    x_pix = idx_f - y_pix * w
    y_feat = jnp.clip(y_pix / 14.0, 0.0, float(h_feat - 1))
    x_feat = jnp.clip(x_pix / 14.0, 0.0, float(w_feat - 1))
    P = _PATCH * _PATCH
    p = lax.broadcasted_iota(jnp.int32, (1, 1, P), 2)
    oy = ((p // _PATCH) - _half).astype(jnp.float32) * _DILATION
    ox = ((p % _PATCH) - _half).astype(jnp.float32) * _DILATION
    y_c = jnp.clip(y_feat[:, :, None] + oy, 0.0, float(h_feat - 1))
    x_c = jnp.clip(x_feat[:, :, None] + ox, 0.0, float(w_feat - 1))
    y_i = jnp.round(y_c).astype(jnp.int32)
    x_i = jnp.round(x_c).astype(jnp.int32)
    b = lax.broadcasted_iota(jnp.int32, y_i.shape, 0)
    out_ref[...] = b * (h_feat * w_feat) + y_i * w_feat + x_i


def _compute_rows(sampling_idx, widths, h_feat, w_feat):
    B, R = sampling_idx.shape
    P = _PATCH * _PATCH
    wf = jnp.asarray(widths, jnp.float32).reshape(1, 1)
    rows = pl.pallas_call(
        functools.partial(_rows_tc_kernel, h_feat=h_feat, w_feat=w_feat),
        out_shape=jax.ShapeDtypeStruct((B, R, P), jnp.int32),
        in_specs=[
            pl.BlockSpec(memory_space=pltpu.SMEM),
            pl.BlockSpec(memory_space=pltpu.VMEM),
        ],
        out_specs=pl.BlockSpec(memory_space=pltpu.VMEM),
    )(wf, sampling_idx)
    return rows.reshape(B * R * P)


def _sc_gather(table, rows, B, R, P):
    D = table.shape[1]
    rays = B * R
    NR = rays // _NW
    wpb = _NW // B
    PP = P + 1
    NCHUNK = PP // 2

    idx_t = rows.reshape(B, wpb, NR, P).transpose(0, 1, 3, 2)
    idx_t = jnp.concatenate(
        [idx_t, jnp.zeros((B, wpb, 1, NR), jnp.int32)], axis=2)
    idx_all = idx_t.reshape(_NW, NCHUNK, 2 * NR)

    mesh = plsc.VectorSubcoreMesh(
        core_axis_name="c", subcore_axis_name="s",
        num_cores=_NC, num_subcores=_NS)

    @functools.partial(
        pl.kernel,
        out_type=jax.ShapeDtypeStruct((B, R, P * D), jnp.float32),
        mesh=mesh,
        scratch_types=[
            pltpu.VMEM((NCHUNK, 2 * NR), jnp.int32),
            [pltpu.VMEM((2 * NR, D), jnp.float32) for _ in range(2)],
            [pltpu.SemaphoreType.DMA for _ in range(2)],
            [pltpu.SemaphoreType.DMA for _ in range(2)],
        ],
    )
    def k(table_hbm, rows_hbm, out_hbm, idx_v, bufs, gsems, wsems):
        wid = lax.axis_index("s") * _NC + lax.axis_index("c")
        bi = wid // wpb
        ray0 = (wid % wpb) * NR

        pltpu.sync_copy(rows_hbm.at[wid], idx_v)

        def start_gather(c, b):
            pltpu.async_copy(table_hbm.at[idx_v.at[c]], bufs[b], gsems[b])

        def wait_gather(b):
            pltpu.make_async_copy(
                table_hbm.at[pl.ds(0, 2 * NR)], bufs[b], gsems[b]).wait()

        def start_write(c, b, both):
            pltpu.async_copy(
                bufs[b].at[pl.ds(0, NR)],
                out_hbm.at[bi, pl.ds(ray0, NR), pl.ds(2 * c * D, D)],
                wsems[b])
            if both:
                pltpu.async_copy(
                    bufs[b].at[pl.ds(NR, NR)],
                    out_hbm.at[bi, pl.ds(ray0, NR), pl.ds((2 * c + 1) * D, D)],
                    wsems[b])

        def wait_write(b, nwrites):
            pltpu.make_async_copy(
                bufs[b].at[pl.ds(0, nwrites * NR)],
                out_hbm.at[0, pl.ds(0, nwrites * NR), pl.ds(0, D)],
                wsems[b]).wait()

        def body(j, carry):
            for b in range(2):
                c = j * 2 + b

                @pl.when(j > 0)
                def _():
                    wait_write(b, 2)

                start_gather(c, b)
                wait_gather(b)
                start_write(c, b, both=True)
            return carry

        lax.fori_loop(0, (NCHUNK - 1) // 2, body, 0)
        wait_write(0, 2)
        start_gather(NCHUNK - 1, 0)
        wait_gather(0)
        start_write(NCHUNK - 1, 0, both=False)
        wait_write(0, 1)
        wait_write(1, 2)

    return k(table, idx_all)


def kernel(feature_maps, sampling_idx, heights, widths):
    B, C, H_feat, W_feat = feature_maps.shape
    R = sampling_idx.shape[1]
    P = _PATCH * _PATCH
    table = feature_maps.transpose(0, 2, 3, 1).reshape(B * H_feat * W_feat, C)
    rows = _compute_rows(sampling_idx, widths, H_feat, W_feat)
    return _sc_gather(table, rows, B, R, P)


# speedup vs baseline: 1.4961x; 1.4961x over previous
import functools

import numpy as np
import jax
import jax.numpy as jnp
from jax import lax
from jax.experimental import pallas as pl
from jax.experimental.pallas import tpu as pltpu
from jax.experimental.pallas import tpu_sc as plsc

_PATCH = 7
_DILATION = 2
_NC, _NS = 2, 16
_NW = _NC * _NS

_half = (_PATCH - 1) // 2


def _rows_tc_kernel(w_ref, idx_ref, out_ref, *, h_feat, w_feat):
    w = w_ref[0, 0]
    idx_f = idx_ref[...].astype(jnp.float32)
    y_pix = jnp.floor(idx_f / w)
    x_pix = idx_f - y_pix * w
    y_feat = jnp.clip(y_pix / 14.0, 0.0, float(h_feat - 1))
    x_feat = jnp.clip(x_pix / 14.0, 0.0, float(w_feat - 1))
    P = _PATCH * _PATCH
    p = lax.broadcasted_iota(jnp.int32, (1, 1, P), 2)
    oy = ((p // _PATCH) - _half).astype(jnp.float32) * _DILATION
    ox = ((p % _PATCH) - _half).astype(jnp.float32) * _DILATION
    y_c = jnp.clip(y_feat[:, :, None] + oy, 0.0, float(h_feat - 1))
    x_c = jnp.clip(x_feat[:, :, None] + ox, 0.0, float(w_feat - 1))
    y_i = jnp.round(y_c).astype(jnp.int32)
    x_i = jnp.round(x_c).astype(jnp.int32)
    b = lax.broadcasted_iota(jnp.int32, y_i.shape, 0)
    out_ref[...] = b * (h_feat * w_feat) + y_i * w_feat + x_i


def _compute_rows(sampling_idx, widths, h_feat, w_feat):
    B, R = sampling_idx.shape
    P = _PATCH * _PATCH
    wf = jnp.asarray(widths, jnp.float32).reshape(1, 1)
    rows = pl.pallas_call(
        functools.partial(_rows_tc_kernel, h_feat=h_feat, w_feat=w_feat),
        out_shape=jax.ShapeDtypeStruct((B, R, P), jnp.int32),
        in_specs=[
            pl.BlockSpec(memory_space=pltpu.SMEM),
            pl.BlockSpec(memory_space=pltpu.VMEM),
        ],
        out_specs=pl.BlockSpec(memory_space=pltpu.VMEM),
    )(wf, sampling_idx)
    return rows.reshape(B * R * P)


def _sc_gather(table, rows, B, R, P):
    D = table.shape[1]
    rays = B * R
    NR = rays // _NW
    wpb = _NW // B

    idx_all = rows.reshape(B, wpb, NR, P).transpose(0, 1, 3, 2).reshape(
        _NW, P, NR)

    mesh = plsc.VectorSubcoreMesh(
        core_axis_name="c", subcore_axis_name="s",
        num_cores=_NC, num_subcores=_NS)

    @functools.partial(
        pl.kernel,
        out_type=jax.ShapeDtypeStruct((B, R, P * D), jnp.float32),
        mesh=mesh,
        scratch_types=[
            pltpu.VMEM((P, NR), jnp.int32),
            [pltpu.VMEM((NR, D), jnp.float32) for _ in range(4)],
            [pltpu.SemaphoreType.DMA for _ in range(4)],
            [pltpu.SemaphoreType.DMA for _ in range(4)],
        ],
    )
    def k(table_hbm, rows_hbm, out_hbm, idx_v, bufs, gsems, wsems):
        wid = lax.axis_index("s") * _NC + lax.axis_index("c")
        bi = wid // wpb
        ray0 = (wid % wpb) * NR

        pltpu.sync_copy(rows_hbm.at[wid], idx_v)

        def start_gather(p, b):
            pltpu.async_copy(table_hbm.at[idx_v.at[p]], bufs[b], gsems[b])

        def wait_gather(b):
            pltpu.make_async_copy(
                table_hbm.at[pl.ds(0, NR)], bufs[b], gsems[b]).wait()

        def start_write(p, b):
            pltpu.async_copy(
                bufs[b],
                out_hbm.at[bi, pl.ds(ray0, NR), pl.ds(p * D, D)],
                wsems[b])

        def wait_write(b):
            pltpu.make_async_copy(
                bufs[b],
                out_hbm.at[bi, pl.ds(ray0, NR), pl.ds(0, D)],
                wsems[b]).wait()

        start_gather(0, 0)
        start_gather(1, 1)

        def body(j, carry):
            for b in range(4):
                p = j * 4 + b
                b2 = (b + 2) % 4
                if b < 2:
                    @pl.when(j > 0)
                    def _():
                        wait_write(b2)

                    start_gather(p + 2, b2)
                else:
                    wait_write(b2)

                    if b == 2:
                        start_gather(p + 2, b2)
                    else:
                        @pl.when(j < (P // 4) - 1)
                        def _():
                            start_gather(p + 2, b2)

                wait_gather(b)
                start_write(p, b)
            return carry

        lax.fori_loop(0, P // 4, body, 0)
        wait_write(2)
        wait_gather(0)
        start_write(P - 1, 0)
        wait_write(3)
        wait_write(0)

    return k(table, idx_all)


def kernel(feature_maps, sampling_idx, heights, widths):
    B, C, H_feat, W_feat = feature_maps.shape
    R = sampling_idx.shape[1]
    P = _PATCH * _PATCH
    table = feature_maps.transpose(0, 2, 3, 1).reshape(B * H_feat * W_feat, C)
    rows = _compute_rows(sampling_idx, widths, H_feat, W_feat)
    return _sc_gather(table, rows, B, R, P)
